# Initial kernel scaffold; baseline (speedup 1.0000x reference)
import functools
import math

import jax
import jax.numpy as jnp
from jax.experimental import pallas as pl
from jax.experimental.pallas import tpu as pltpu

B = 4
S = 2048
D_MODEL = 1024
RANK = 64
N_COMPRESS = 64
N_KNOWLEDGE = 8192
K_KNOW = 8
TOPK_C = 16

TS = 128
NEG = -1e30
BIGIDX = jnp.int32(2**30)


def _mix_kernel(w_ref, idx_ref, cn_ref, out_ref):
    iota_n = jax.lax.broadcasted_iota(jnp.int32, (B, N_COMPRESS), 1)
    c = jnp.zeros((B, N_COMPRESS), jnp.float32)
    for k in range(TOPK_C):
        c = c + w_ref[:, k:k + 1] * (idx_ref[:, k:k + 1] == iota_n).astype(jnp.float32)
    out_ref[...] = jnp.dot(c, cn_ref[...], preferred_element_type=jnp.float32)


def _main_kernel(x_ref, sc_ref, k_ref, v_ref, out_ref, idx_ref, w_ref):
    x = x_ref[0]
    shared_c = sc_ref[0]
    q = jnp.dot(x, shared_c, preferred_element_type=jnp.float32)
    s0 = jax.lax.dot_general(
        q, k_ref[...], (((1,), (1,)), ((), ())),
        preferred_element_type=jnp.float32) * (1.0 / math.sqrt(RANK))

    iota = jax.lax.broadcasted_iota(jnp.int32, (TS, N_KNOWLEDGE), 1)
    s = s0
    vals = []
    idxs = []
    for _ in range(K_KNOW):
        m = jnp.max(s, axis=1, keepdims=True)
        cand = jnp.where(s == m, iota, BIGIDX)
        a = jnp.min(cand, axis=1, keepdims=True)
        vals.append(m)
        idxs.append(a)
        s = jnp.where(iota == a, NEG, s)

    v8 = jnp.concatenate(vals, axis=1)
    i8 = jnp.concatenate(idxs, axis=1)
    e8 = jnp.exp(v8 - v8[:, 0:1])
    denom = jnp.sum(e8, axis=1, keepdims=True)
    w8 = e8 / denom

    w_full = jnp.where(s < s0, jnp.exp(s0 - v8[:, 0:1]), 0.0) / denom
    out_ref[0] = jnp.dot(w_full, v_ref[...], preferred_element_type=jnp.float32)
    idx_ref[0] = i8
    w_ref[0] = w8


def kernel(x, memory_topk_w, memory_topk_idx, compress_neurons, knowledge_K, knowledge_V):
    cn2 = compress_neurons.reshape(N_COMPRESS, D_MODEL * RANK)
    shared_flat = pl.pallas_call(
        _mix_kernel,
        grid=(16,),
        in_specs=[
            pl.BlockSpec((B, TOPK_C), lambda i: (0, 0)),
            pl.BlockSpec((B, TOPK_C), lambda i: (0, 0)),
            pl.BlockSpec((N_COMPRESS, D_MODEL * RANK // 16), lambda i: (0, i)),
        ],
        out_specs=pl.BlockSpec((B, D_MODEL * RANK // 16), lambda i: (0, i)),
        out_shape=jax.ShapeDtypeStruct((B, D_MODEL * RANK), jnp.float32),
    )(memory_topk_w, memory_topk_idx, cn2)
    shared_compress = shared_flat.reshape(B, D_MODEL, RANK)

    out, topk_idx, weights = pl.pallas_call(
        _main_kernel,
        grid=(B, S // TS),
        in_specs=[
            pl.BlockSpec((1, TS, D_MODEL), lambda b, s: (b, s, 0)),
            pl.BlockSpec((1, D_MODEL, RANK), lambda b, s: (b, 0, 0)),
            pl.BlockSpec((N_KNOWLEDGE, RANK), lambda b, s: (0, 0)),
            pl.BlockSpec((N_KNOWLEDGE, D_MODEL), lambda b, s: (0, 0)),
        ],
        out_specs=[
            pl.BlockSpec((1, TS, D_MODEL), lambda b, s: (b, s, 0)),
            pl.BlockSpec((1, TS, K_KNOW), lambda b, s: (b, s, 0)),
            pl.BlockSpec((1, TS, K_KNOW), lambda b, s: (b, s, 0)),
        ],
        out_shape=[
            jax.ShapeDtypeStruct((B, S, D_MODEL), jnp.float32),
            jax.ShapeDtypeStruct((B, S, K_KNOW), jnp.int32),
            jax.ShapeDtypeStruct((B, S, K_KNOW), jnp.float32),
        ],
    )(x, memory := shared_compress, knowledge_K, knowledge_V)
    return (out, topk_idx, weights)


# speedup vs baseline: 21.3542x; 21.3542x over previous
import functools
import math

import jax
import jax.numpy as jnp
from jax.experimental import pallas as pl
from jax.experimental.pallas import tpu as pltpu

B = 4
S = 2048
D_MODEL = 1024
RANK = 64
N_COMPRESS = 64
N_KNOWLEDGE = 8192
K_KNOW = 8
TOPK_C = 16

TS = 128
NEG = -1e30
BIGIDX = 2**30


def _mix_kernel(w_ref, idx_ref, cn_ref, out_ref):
    iota3 = jax.lax.broadcasted_iota(jnp.int32, (B, TOPK_C, N_COMPRESS), 2)
    oh = (iota3 == idx_ref[...][..., None]).astype(jnp.bfloat16)
    cn_bf = cn_ref[...].astype(jnp.bfloat16)
    w_bf = w_ref[...].astype(jnp.bfloat16)
    for b in range(B):
        rows = jnp.dot(oh[b], cn_bf, preferred_element_type=jnp.float32)
        shb = jnp.dot(w_bf[b:b + 1], rows.astype(jnp.bfloat16),
                      preferred_element_type=jnp.float32)
        out_ref[b:b + 1, :] = shb


def _main_kernel(x_ref, sc_ref, k_ref, v_ref, out_ref, idx_ref, w_ref):
    x = x_ref[0].astype(jnp.bfloat16)
    shared_c = sc_ref[0].astype(jnp.bfloat16)
    q = jnp.dot(x, shared_c, preferred_element_type=jnp.float32)
    s0 = jax.lax.dot_general(
        q.astype(jnp.bfloat16), k_ref[...].astype(jnp.bfloat16),
        (((1,), (1,)), ((), ())),
        preferred_element_type=jnp.float32) * (1.0 / math.sqrt(RANK))

    iota = jax.lax.broadcasted_iota(jnp.int32, (TS, N_KNOWLEDGE), 1)
    s = s0
    vals = []
    idxs = []
    for _ in range(K_KNOW):
        m = jnp.max(s, axis=1, keepdims=True)
        cand = jnp.where(s == m, iota, BIGIDX)
        a = jnp.min(cand, axis=1, keepdims=True)
        vals.append(m)
        idxs.append(a)
        s = jnp.where(iota == a, NEG, s)

    v8 = jnp.concatenate(vals, axis=1)
    i8 = jnp.concatenate(idxs, axis=1)
    e8 = jnp.exp(v8 - v8[:, 0:1])
    denom = jnp.sum(e8, axis=1, keepdims=True)
    w8 = e8 / denom

    w_full = jnp.where(s < s0, jnp.exp(s0 - v8[:, 0:1]), 0.0) / denom
    out_ref[0] = jnp.dot(w_full, v_ref[...], preferred_element_type=jnp.float32)
    idx_ref[0] = i8
    w_ref[0] = w8


def kernel(x, memory_topk_w, memory_topk_idx, compress_neurons, knowledge_K, knowledge_V):
    cn2 = compress_neurons.reshape(N_COMPRESS, D_MODEL * RANK)
    shared_flat = pl.pallas_call(
        _mix_kernel,
        grid=(16,),
        in_specs=[
            pl.BlockSpec((B, TOPK_C), lambda i: (0, 0)),
            pl.BlockSpec((B, TOPK_C), lambda i: (0, 0)),
            pl.BlockSpec((N_COMPRESS, D_MODEL * RANK // 16), lambda i: (0, i)),
        ],
        out_specs=pl.BlockSpec((B, D_MODEL * RANK // 16), lambda i: (0, i)),
        out_shape=jax.ShapeDtypeStruct((B, D_MODEL * RANK), jnp.float32),
    )(memory_topk_w, memory_topk_idx, cn2)
    shared_compress = shared_flat.reshape(B, D_MODEL, RANK)

    out, topk_idx, weights = pl.pallas_call(
        _main_kernel,
        grid=(B, S // TS),
        in_specs=[
            pl.BlockSpec((1, TS, D_MODEL), lambda b, s: (b, s, 0)),
            pl.BlockSpec((1, D_MODEL, RANK), lambda b, s: (b, 0, 0)),
            pl.BlockSpec((N_KNOWLEDGE, RANK), lambda b, s: (0, 0)),
            pl.BlockSpec((N_KNOWLEDGE, D_MODEL), lambda b, s: (0, 0)),
        ],
        out_specs=[
            pl.BlockSpec((1, TS, D_MODEL), lambda b, s: (b, s, 0)),
            pl.BlockSpec((1, TS, K_KNOW), lambda b, s: (b, s, 0)),
            pl.BlockSpec((1, TS, K_KNOW), lambda b, s: (b, s, 0)),
        ],
        out_shape=[
            jax.ShapeDtypeStruct((B, S, D_MODEL), jnp.float32),
            jax.ShapeDtypeStruct((B, S, K_KNOW), jnp.int32),
            jax.ShapeDtypeStruct((B, S, K_KNOW), jnp.float32),
        ],
    )(x, shared_compress, knowledge_K, knowledge_V)
    return (out, topk_idx, weights)
